# baseline (speedup 1.0000x reference)
import functools

import jax
import jax.numpy as jnp
from jax import lax
from jax.experimental import pallas as pl
from jax.experimental.pallas import tpu as pltpu
from jax.experimental.pallas import tpu_sc as plsc

_N = 10000
_E = 320000
_D = 128
_H = 1024
_C = 153

_NSC = 2
_NTILE = 16
_NP = 10240
_HALF = _NP // _NSC

_EPT = _E // _NTILE
_ECH = 2000


def _make_segsum(cm, r):
    fe = 128 // cm
    npass = _HALF // r
    app = 2 * fe + 32
    trash = 2 * fe + 16
    slabr = r + 16
    zsh = slabr // _NTILE
    wsh = r // _NTILE
    mesh = plsc.VectorSubcoreMesh(core_axis_name="c", subcore_axis_name="s")

    @functools.partial(
        pl.kernel,
        out_type=jax.ShapeDtypeStruct((_NP, cm, 128), jnp.float32),
        mesh=mesh,
        scratch_types=[
            pltpu.VMEM((_ECH,), jnp.int32),
            pltpu.VMEM((_ECH,), jnp.int32),
            pltpu.VMEM((app,), jnp.int32),
            pltpu.VMEM((app,), jnp.int32),
            [pltpu.VMEM((fe,), jnp.int32)] * 3,
            [pltpu.VMEM((fe,), jnp.int32)] * 3,
            [pltpu.VMEM((fe, cm, 128), jnp.float32)] * 3,
            pltpu.VMEM((1, cm, 128), jnp.float32),
            pltpu.VMEM((fe,), jnp.int32),
            pltpu.VMEM_SHARED((slabr, cm, 128), jnp.float32),
            [pltpu.SemaphoreType.DMA] * 3,
            [pltpu.SemaphoreType.DMA] * 3,
        ],
        compiler_params=pltpu.CompilerParams(needs_layout_passes=False),
    )
    def segsum(table, srcv, dstv, out, src_c, dst_c, psrc, pdst,
               fs, fd, stg, zrow, fdd, slab, sem_g, sem_s):
        cid = lax.axis_index("c")
        sid = lax.axis_index("s")
        ebase = sid * _EPT

        def _zcol(cc, _):
            zrow[0, cc // 8, pl.ds((cc % 8) * 16, 16)] = jnp.zeros(
                (16,), jnp.float32)
            return 0
        lax.fori_loop(0, cm * 8, _zcol, 0)

        def fire_k(k):
            pltpu.make_async_copy(stg[k], slab.at[fd[k]], sem_s[k]).wait()
            for j in range(fe // 16):
                fs[k][pl.ds(j * 16, 16)] = psrc[pl.ds(j * 16, 16)]
                fd[k][pl.ds(j * 16, 16)] = pdst[pl.ds(j * 16, 16)]
            pltpu.async_copy(table.at[fs[k]], stg[k], sem_g[k])
            k2 = (k + 1) % 3
            pltpu.make_async_copy(table.at[fs[k2]], stg[k2], sem_g[k2]).wait()
            pltpu.async_copy(stg[k2], slab.at[fd[k2]], sem_s[k2], add=True)

        def fire_parity(nfr):
            m3 = nfr % 3
            lax.cond(m3 == 0, lambda: fire_k(0),
                     lambda: lax.cond(m3 == 1, lambda: fire_k(1),
                                      lambda: fire_k(2)))

        def retire_k(k):
            pltpu.make_async_copy(table.at[fs[k]], stg[k], sem_g[k]).wait()
            pltpu.async_copy(stg[k], slab.at[fd[k]], sem_s[k], add=True)

        def retire_parity(idx3):
            lax.cond(idx3 == 0, lambda: retire_k(0),
                     lambda: lax.cond(idx3 == 1, lambda: retire_k(1),
                                      lambda: retire_k(2)))

        def pass_body(p, _):
            wbase = cid * _HALF + p * r
            def _z1(k, _):
                pltpu.sync_copy(zrow, slab.at[pl.ds(sid * zsh + k, 1)])
                return 0
            lax.fori_loop(0, zsh, _z1, 0)
            plsc.subcore_barrier()

            for kk in range(fe // 16):
                z16 = jnp.zeros((16,), jnp.int32)
                d16 = jnp.full((16,), r, jnp.int32)
                fs[1][pl.ds(kk * 16, 16)] = z16
                fs[2][pl.ds(kk * 16, 16)] = z16
                fd[1][pl.ds(kk * 16, 16)] = d16
                fd[2][pl.ds(kk * 16, 16)] = d16
                fdd[pl.ds(kk * 16, 16)] = d16
            pltpu.async_copy(table.at[fs[1]], stg[1], sem_g[1])
            pltpu.async_copy(table.at[fs[2]], stg[2], sem_g[2])
            pltpu.async_copy(stg[0], slab.at[fdd], sem_s[0], add=True)

            def chunk_body(jc, carry):
                pltpu.sync_copy(srcv.at[pl.ds(ebase + jc * _ECH, _ECH)], src_c)
                pltpu.sync_copy(dstv.at[pl.ds(ebase + jc * _ECH, _ECH)], dst_c)

                def vec_body(jv, carry):
                    nf, nfire = carry
                    s16 = src_c[pl.ds(jv * 16, 16)]
                    d16 = dst_c[pl.ds(jv * 16, 16)]
                    dloc = d16 - wbase
                    m = dloc.astype(jnp.uint32) < jnp.uint32(r)
                    csum = jnp.cumsum(jnp.where(m, 1, 0))
                    pos = jnp.where(m, nf + csum - 1, trash)
                    plsc.store_scatter(psrc, [pos], s16)
                    plsc.store_scatter(pdst, [pos], dloc)
                    nf2 = nf + jnp.max(csum)

                    def do_fire(c):
                        v, nfr = c
                        fire_parity(nfr)
                        psrc[pl.ds(0, 16)] = psrc[pl.ds(fe, 16)]
                        pdst[pl.ds(0, 16)] = pdst[pl.ds(fe, 16)]
                        return v - fe, nfr + 1

                    return lax.cond(nf2 >= fe, do_fire, lambda c: c,
                                    (nf2, nfire))

                return lax.fori_loop(0, _ECH // 16, vec_body, carry)

            nf, nfire = lax.fori_loop(0, _EPT // _ECH, chunk_body, (0, 0))

            for kk in range(fe // 16):
                psrc[pl.ds(nf + kk * 16, 16)] = jnp.zeros((16,), jnp.int32)
                pdst[pl.ds(nf + kk * 16, 16)] = jnp.full((16,), r, jnp.int32)
            fire_parity(nfire)
            retire_parity((nfire + 2) % 3)
            retire_parity(nfire % 3)
            for k in range(3):
                pltpu.make_async_copy(stg[k], slab.at[fd[k]], sem_s[k]).wait()
            plsc.subcore_barrier()

            pltpu.sync_copy(slab.at[pl.ds(sid * wsh, wsh)],
                            out.at[pl.ds(wbase + sid * wsh, wsh)])
            plsc.subcore_barrier()
            return 0

        lax.fori_loop(0, npass, pass_body, 0)

    return segsum


_segsum_l1 = _make_segsum(cm=2, r=2560)
_segsum_l2 = _make_segsum(cm=8, r=1024)

_ROWS_BLK = 400
_GRID = _N // _ROWS_BLK


def _elu(z):
    return jnp.where(z > 0, z, jnp.exp(jnp.minimum(z, 0.0)) - 1.0)


def _tc1_body(s_ref, x_ref, wl_ref, b_ref, wr_ref, h_ref):
    s = s_ref[...]
    rcp = 1.0 / jnp.maximum(s[:, 128:129], 1.0)
    mean = s[:, :128] * rcp
    z = (jnp.dot(mean, wl_ref[...], preferred_element_type=jnp.float32)
         + b_ref[...]
         + jnp.dot(x_ref[...], wr_ref[...], preferred_element_type=jnp.float32))
    h_ref[...] = _elu(z)


def _tc1(sums1, x, W1l, b1, W1r):
    return pl.pallas_call(
        _tc1_body,
        grid=(_GRID,),
        in_specs=[
            pl.BlockSpec((_ROWS_BLK, 256), lambda i: (i, 0)),
            pl.BlockSpec((_ROWS_BLK, _D), lambda i: (i, 0)),
            pl.BlockSpec((_D, _H), lambda i: (0, 0)),
            pl.BlockSpec((1, _H), lambda i: (0, 0)),
            pl.BlockSpec((_D, _H), lambda i: (0, 0)),
        ],
        out_specs=pl.BlockSpec((_ROWS_BLK, _H), lambda i: (i, 0)),
        out_shape=jax.ShapeDtypeStruct((_N, _H), jnp.float32),
    )(sums1, x, W1l, b1, W1r)


def _tc2_body(s2_ref, s1_ref, h_ref, wl_ref, b_ref, wr_ref, wc_ref, bc_ref,
              o_ref):
    rcp = 1.0 / jnp.maximum(s1_ref[:, 128:129], 1.0)
    mean = s2_ref[...] * rcp
    z = (jnp.dot(mean, wl_ref[...], preferred_element_type=jnp.float32)
         + b_ref[...]
         + jnp.dot(h_ref[...], wr_ref[...], preferred_element_type=jnp.float32))
    h2 = _elu(z)
    o_ref[...] = jnp.dot(h2, wc_ref[...], preferred_element_type=jnp.float32) + bc_ref[...]


def _tc2(sums2, sums1, h, W2l, b2, W2r, Wcp, bcp):
    return pl.pallas_call(
        _tc2_body,
        grid=(_GRID,),
        in_specs=[
            pl.BlockSpec((_ROWS_BLK, _H), lambda i: (i, 0)),
            pl.BlockSpec((_ROWS_BLK, 256), lambda i: (i, 0)),
            pl.BlockSpec((_ROWS_BLK, _H), lambda i: (i, 0)),
            pl.BlockSpec((_H, _H), lambda i: (0, 0)),
            pl.BlockSpec((1, _H), lambda i: (0, 0)),
            pl.BlockSpec((_H, _H), lambda i: (0, 0)),
            pl.BlockSpec((_H, 256), lambda i: (0, 0)),
            pl.BlockSpec((1, 256), lambda i: (0, 0)),
        ],
        out_specs=pl.BlockSpec((_ROWS_BLK, 256), lambda i: (i, 0)),
        out_shape=jax.ShapeDtypeStruct((_N, 256), jnp.float32),
    )(sums2, sums1, h, W2l, b2, W2r, Wcp, bcp)


def kernel(x, edge_index, W1l, b1, W1r, W2l, b2, W2r, Wc, bc):
    src = edge_index[0].astype(jnp.int32)
    dst = edge_index[1].astype(jnp.int32)

    x_aug = jnp.concatenate(
        [x, jnp.ones((_N, 1), jnp.float32), jnp.zeros((_N, 127), jnp.float32)],
        axis=1)

    sums1 = _segsum_l1(x_aug.reshape(_N, 2, 128), src, dst)
    sums1 = sums1.reshape(_NP, 256)[:_N]
    h = _tc1(sums1, x, W1l, b1.reshape(1, _H), W1r)
    sums2 = _segsum_l2(h.reshape(_N, 8, 128), src, dst)
    sums2 = sums2.reshape(_NP, _H)[:_N]
    Wcp = jnp.pad(Wc, ((0, 0), (0, 256 - _C)))
    bcp = jnp.pad(bc, (0, 256 - _C)).reshape(1, 256)
    out = _tc2(sums2, sums1, h, W2l, b2.reshape(1, _H), W2r, Wcp, bcp)
    return out[:, :_C]


# speedup vs baseline: 1.0697x; 1.0697x over previous
import functools

import jax
import jax.numpy as jnp
from jax import lax
from jax.experimental import pallas as pl
from jax.experimental.pallas import tpu as pltpu
from jax.experimental.pallas import tpu_sc as plsc

_N = 10000
_E = 320000
_D = 128
_H = 1024
_C = 153

_NSC = 2
_NTILE = 16
_NP = 10240
_HALF = _NP // _NSC

_EPT = _E // _NTILE
_ECH = 2000


def _make_segsum(cm):
    fe = 128 // cm
    r = 10240 // cm
    npass = cm // 2
    app = 2 * fe + 32
    trash = 2 * fe + 16
    slabr = r + 16
    zsh = slabr // _NTILE
    wsh = r // _NTILE
    mesh = plsc.VectorSubcoreMesh(core_axis_name="c", subcore_axis_name="s")

    @functools.partial(
        pl.kernel,
        out_type=jax.ShapeDtypeStruct((_NP, cm, 128), jnp.float32),
        mesh=mesh,
        scratch_types=[
            pltpu.VMEM((_ECH,), jnp.int32),
            pltpu.VMEM((_ECH,), jnp.int32),
            pltpu.VMEM((_ECH,), jnp.int32),
            pltpu.VMEM((_ECH,), jnp.int32),
            pltpu.VMEM((app,), jnp.int32),
            pltpu.VMEM((app,), jnp.int32),
            pltpu.VMEM((fe,), jnp.int32),
            pltpu.VMEM((fe,), jnp.int32),
            pltpu.VMEM((fe,), jnp.int32),
            pltpu.VMEM((fe,), jnp.int32),
            pltpu.VMEM((fe, cm, 128), jnp.float32),
            pltpu.VMEM((fe, cm, 128), jnp.float32),
            pltpu.VMEM((1, cm, 128), jnp.float32),
            pltpu.VMEM((fe,), jnp.int32),
            pltpu.VMEM_SHARED((slabr, cm, 128), jnp.float32),
            pltpu.SemaphoreType.DMA,
            pltpu.SemaphoreType.DMA,
            pltpu.SemaphoreType.DMA,
            pltpu.SemaphoreType.DMA,
        ],
        compiler_params=pltpu.CompilerParams(needs_layout_passes=False),
    )
    def segsum(table, srcv, dstv, out, src_c0, dst_c0, src_c1, dst_c1,
               psrc, pdst, fsa, fda, fsb, fdb, stga, stgb, zrow, fdd, slab,
               sem_c, sem_g, sem_sa, sem_sb):
        cid = lax.axis_index("c")
        sid = lax.axis_index("s")
        ebase = sid * _EPT

        def _zcol(cc, _):
            zrow[0, cc // 8, pl.ds((cc % 8) * 16, 16)] = jnp.zeros(
                (16,), jnp.float32)
            return 0
        lax.fori_loop(0, cm * 8, _zcol, 0)

        def fire_static(fs_cur, fd_cur, stg_cur, sem_cur,
                        fs_oth, fd_oth, stg_oth, sem_oth):
            pltpu.make_async_copy(stg_cur, slab.at[fd_cur], sem_cur).wait()
            for j in range(fe // 16):
                fs_cur[pl.ds(j * 16, 16)] = psrc[pl.ds(j * 16, 16)]
                fd_cur[pl.ds(j * 16, 16)] = pdst[pl.ds(j * 16, 16)]
            pltpu.make_async_copy(table.at[fs_oth], stg_oth, sem_g).wait()
            pltpu.async_copy(table.at[fs_cur], stg_cur, sem_g)
            pltpu.async_copy(stg_oth, slab.at[fd_oth], sem_oth, add=True)

        def fire_parity(nfr):
            lax.cond(
                nfr % 2 == 0,
                lambda: fire_static(fsa, fda, stga, sem_sa,
                                    fsb, fdb, stgb, sem_sb),
                lambda: fire_static(fsb, fdb, stgb, sem_sb,
                                    fsa, fda, stga, sem_sa),
            )

        def pass_body(p, _):
            wbase = cid * _HALF + p * r
            def _z1(k, _):
                pltpu.sync_copy(zrow, slab.at[pl.ds(sid * zsh + k, 1)])
                return 0
            lax.fori_loop(0, zsh, _z1, 0)
            plsc.subcore_barrier()

            for kk in range(fe // 16):
                z16 = jnp.zeros((16,), jnp.int32)
                d16 = jnp.full((16,), r, jnp.int32)
                fsb[pl.ds(kk * 16, 16)] = z16
                fdb[pl.ds(kk * 16, 16)] = d16
                fdd[pl.ds(kk * 16, 16)] = d16
            pltpu.async_copy(table.at[fsb], stgb, sem_g)
            pltpu.async_copy(stga, slab.at[fdd], sem_sa, add=True)

            pltpu.async_copy(srcv.at[pl.ds(ebase, _ECH)], src_c0, sem_c)
            pltpu.async_copy(dstv.at[pl.ds(ebase, _ECH)], dst_c0, sem_c)

            def scan_chunk(jc, carry, src_c, dst_c, src_n, dst_n):
                pltpu.make_async_copy(srcv.at[pl.ds(ebase, _ECH)],
                                      src_c, sem_c).wait()
                pltpu.make_async_copy(dstv.at[pl.ds(ebase, _ECH)],
                                      dst_c, sem_c).wait()
                nxt = jnp.minimum(ebase + (jc + 1) * _ECH, _E - _ECH)
                pltpu.async_copy(srcv.at[pl.ds(nxt, _ECH)], src_n, sem_c)
                pltpu.async_copy(dstv.at[pl.ds(nxt, _ECH)], dst_n, sem_c)

                def vec_body(jv, carry):
                    nf, nfire = carry
                    s16 = src_c[pl.ds(jv * 16, 16)]
                    d16 = dst_c[pl.ds(jv * 16, 16)]
                    dloc = d16 - wbase
                    m = dloc.astype(jnp.uint32) < jnp.uint32(r)
                    csum = jnp.cumsum(jnp.where(m, 1, 0))
                    pos = jnp.where(m, nf + csum - 1, trash)
                    plsc.store_scatter(psrc, [pos], s16)
                    plsc.store_scatter(pdst, [pos], dloc)
                    nf2 = nf + jnp.max(csum)

                    def do_fire(c):
                        v, nfr = c
                        fire_parity(nfr)
                        psrc[pl.ds(0, 16)] = psrc[pl.ds(fe, 16)]
                        pdst[pl.ds(0, 16)] = pdst[pl.ds(fe, 16)]
                        return v - fe, nfr + 1

                    return lax.cond(nf2 >= fe, do_fire, lambda c: c,
                                    (nf2, nfire))

                return lax.fori_loop(0, _ECH // 16, vec_body, carry)

            def chunk_body(jc, carry):
                return lax.cond(
                    jc % 2 == 0,
                    lambda c: scan_chunk(jc, c, src_c0, dst_c0,
                                         src_c1, dst_c1),
                    lambda c: scan_chunk(jc, c, src_c1, dst_c1,
                                         src_c0, dst_c0),
                    carry)

            nf, nfire = lax.fori_loop(0, _EPT // _ECH, chunk_body, (0, 0))
            pltpu.make_async_copy(srcv.at[pl.ds(ebase, _ECH)],
                                  src_c0, sem_c).wait()
            pltpu.make_async_copy(dstv.at[pl.ds(ebase, _ECH)],
                                  dst_c0, sem_c).wait()

            for kk in range(fe // 16):
                psrc[pl.ds(nf + kk * 16, 16)] = jnp.zeros((16,), jnp.int32)
                pdst[pl.ds(nf + kk * 16, 16)] = jnp.full((16,), r, jnp.int32)
            fire_parity(nfire)
            lax.cond(
                nfire % 2 == 0,
                lambda: (pltpu.make_async_copy(table.at[fsa], stga,
                                               sem_g).wait(),
                         pltpu.async_copy(stga, slab.at[fda], sem_sa,
                                          add=True))[0],
                lambda: (pltpu.make_async_copy(table.at[fsb], stgb,
                                               sem_g).wait(),
                         pltpu.async_copy(stgb, slab.at[fdb], sem_sb,
                                          add=True))[0],
            )
            pltpu.make_async_copy(stga, slab.at[fda], sem_sa).wait()
            pltpu.make_async_copy(stgb, slab.at[fdb], sem_sb).wait()
            plsc.subcore_barrier()

            pltpu.sync_copy(slab.at[pl.ds(sid * wsh, wsh)],
                            out.at[pl.ds(wbase + sid * wsh, wsh)])
            plsc.subcore_barrier()
            return 0

        lax.fori_loop(0, npass, pass_body, 0)

    return segsum


_segsum_l1 = _make_segsum(cm=2)
_segsum_l2 = _make_segsum(cm=8)

_ROWS_BLK = 400
_GRID = _N // _ROWS_BLK


def _elu(z):
    return jnp.where(z > 0, z, jnp.exp(jnp.minimum(z, 0.0)) - 1.0)


def _tc1_body(s_ref, x_ref, wl_ref, b_ref, wr_ref, h_ref):
    s = s_ref[...]
    rcp = 1.0 / jnp.maximum(s[:, 128:129], 1.0)
    mean = s[:, :128] * rcp
    z = (jnp.dot(mean, wl_ref[...], preferred_element_type=jnp.float32)
         + b_ref[...]
         + jnp.dot(x_ref[...], wr_ref[...], preferred_element_type=jnp.float32))
    h_ref[...] = _elu(z)


def _tc1(sums1, x, W1l, b1, W1r):
    return pl.pallas_call(
        _tc1_body,
        grid=(_GRID,),
        in_specs=[
            pl.BlockSpec((_ROWS_BLK, 256), lambda i: (i, 0)),
            pl.BlockSpec((_ROWS_BLK, _D), lambda i: (i, 0)),
            pl.BlockSpec((_D, _H), lambda i: (0, 0)),
            pl.BlockSpec((1, _H), lambda i: (0, 0)),
            pl.BlockSpec((_D, _H), lambda i: (0, 0)),
        ],
        out_specs=pl.BlockSpec((_ROWS_BLK, _H), lambda i: (i, 0)),
        out_shape=jax.ShapeDtypeStruct((_N, _H), jnp.float32),
    )(sums1, x, W1l, b1, W1r)


def _tc2_body(s2_ref, s1_ref, h_ref, wl_ref, b_ref, wr_ref, wc_ref, bc_ref,
              o_ref):
    rcp = 1.0 / jnp.maximum(s1_ref[:, 128:129], 1.0)
    mean = s2_ref[...] * rcp
    z = (jnp.dot(mean, wl_ref[...], preferred_element_type=jnp.float32)
         + b_ref[...]
         + jnp.dot(h_ref[...], wr_ref[...], preferred_element_type=jnp.float32))
    h2 = _elu(z)
    o_ref[...] = jnp.dot(h2, wc_ref[...], preferred_element_type=jnp.float32) + bc_ref[...]


def _tc2(sums2, sums1, h, W2l, b2, W2r, Wcp, bcp):
    return pl.pallas_call(
        _tc2_body,
        grid=(_GRID,),
        in_specs=[
            pl.BlockSpec((_ROWS_BLK, _H), lambda i: (i, 0)),
            pl.BlockSpec((_ROWS_BLK, 256), lambda i: (i, 0)),
            pl.BlockSpec((_ROWS_BLK, _H), lambda i: (i, 0)),
            pl.BlockSpec((_H, _H), lambda i: (0, 0)),
            pl.BlockSpec((1, _H), lambda i: (0, 0)),
            pl.BlockSpec((_H, _H), lambda i: (0, 0)),
            pl.BlockSpec((_H, 256), lambda i: (0, 0)),
            pl.BlockSpec((1, 256), lambda i: (0, 0)),
        ],
        out_specs=pl.BlockSpec((_ROWS_BLK, 256), lambda i: (i, 0)),
        out_shape=jax.ShapeDtypeStruct((_N, 256), jnp.float32),
    )(sums2, sums1, h, W2l, b2, W2r, Wcp, bcp)


def kernel(x, edge_index, W1l, b1, W1r, W2l, b2, W2r, Wc, bc):
    src = edge_index[0].astype(jnp.int32)
    dst = edge_index[1].astype(jnp.int32)

    x_aug = jnp.concatenate(
        [x, jnp.ones((_N, 1), jnp.float32), jnp.zeros((_N, 127), jnp.float32)],
        axis=1)

    sums1 = _segsum_l1(x_aug.reshape(_N, 2, 128), src, dst)
    sums1 = sums1.reshape(_NP, 256)[:_N]
    h = _tc1(sums1, x, W1l, b1.reshape(1, _H), W1r)
    sums2 = _segsum_l2(h.reshape(_N, 8, 128), src, dst)
    sums2 = sums2.reshape(_NP, _H)[:_N]
    Wcp = jnp.pad(Wc, ((0, 0), (0, 256 - _C)))
    bcp = jnp.pad(bc, (0, 256 - _C)).reshape(1, 256)
    out = _tc2(sums2, sums1, h, W2l, b2.reshape(1, _H), W2r, Wcp, bcp)
    return out[:, :_C]


# speedup vs baseline: 1.0697x; 1.0000x over previous
import functools

import jax
import jax.numpy as jnp
from jax import lax
from jax.experimental import pallas as pl
from jax.experimental.pallas import tpu as pltpu
from jax.experimental.pallas import tpu_sc as plsc

_N = 10000
_E = 320000
_D = 128
_H = 1024
_C = 153

_NSC = 2
_NTILE = 16
_NP = 10240
_HALF = _NP // _NSC

_EPT = _E // _NTILE
_ECH = 2000


def _make_segsum(cm):
    fe = 128 // cm
    r = 10240 // cm
    npass = cm // 2
    app = 2 * fe + 32
    trash = 2 * fe + 16
    slabr = r + 16
    zsh = slabr // _NTILE
    wsh = r // _NTILE
    mesh = plsc.VectorSubcoreMesh(core_axis_name="c", subcore_axis_name="s")

    @functools.partial(
        pl.kernel,
        out_type=jax.ShapeDtypeStruct((_NP, cm, 128), jnp.float32),
        mesh=mesh,
        scratch_types=[
            pltpu.VMEM((_ECH,), jnp.int32),
            pltpu.VMEM((_ECH,), jnp.int32),
            pltpu.VMEM((_ECH,), jnp.int32),
            pltpu.VMEM((_ECH,), jnp.int32),
            pltpu.VMEM((app,), jnp.int32),
            pltpu.VMEM((app,), jnp.int32),
            pltpu.VMEM((fe,), jnp.int32),
            pltpu.VMEM((fe,), jnp.int32),
            pltpu.VMEM((fe,), jnp.int32),
            pltpu.VMEM((fe,), jnp.int32),
            pltpu.VMEM((fe, cm, 128), jnp.float32),
            pltpu.VMEM((fe, cm, 128), jnp.float32),
            pltpu.VMEM((1, cm, 128), jnp.float32),
            pltpu.VMEM((fe,), jnp.int32),
            pltpu.VMEM_SHARED((slabr, cm, 128), jnp.float32),
            pltpu.SemaphoreType.DMA,
            pltpu.SemaphoreType.DMA,
            pltpu.SemaphoreType.DMA,
            pltpu.SemaphoreType.DMA,
        ],
        compiler_params=pltpu.CompilerParams(needs_layout_passes=False),
    )
    def segsum(table, srcv, dstv, out, src_c0, dst_c0, src_c1, dst_c1,
               psrc, pdst, fsa, fda, fsb, fdb, stga, stgb, zrow, fdd, slab,
               sem_c, sem_g, sem_sa, sem_sb):
        cid = lax.axis_index("c")
        sid = lax.axis_index("s")
        ebase = sid * _EPT

        def _zcol(cc, _):
            zrow[0, cc // 8, pl.ds((cc % 8) * 16, 16)] = jnp.zeros(
                (16,), jnp.float32)
            return 0
        lax.fori_loop(0, cm * 8, _zcol, 0)

        def fire_static(fs_cur, fd_cur, stg_cur, sem_cur,
                        fs_oth, fd_oth, stg_oth, sem_oth):
            pltpu.make_async_copy(stg_cur, slab.at[fd_cur], sem_cur).wait()
            for j in range(fe // 16):
                fs_cur[pl.ds(j * 16, 16)] = psrc[pl.ds(j * 16, 16)]
                fd_cur[pl.ds(j * 16, 16)] = pdst[pl.ds(j * 16, 16)]
            pltpu.make_async_copy(table.at[fs_oth], stg_oth, sem_g).wait()
            pltpu.async_copy(table.at[fs_cur], stg_cur, sem_g)
            pltpu.async_copy(stg_oth, slab.at[fd_oth], sem_oth, add=True)

        def fire_parity(nfr):
            lax.cond(
                nfr % 2 == 0,
                lambda: fire_static(fsa, fda, stga, sem_sa,
                                    fsb, fdb, stgb, sem_sb),
                lambda: fire_static(fsb, fdb, stgb, sem_sb,
                                    fsa, fda, stga, sem_sa),
            )

        def pass_body(p, _):
            wbase = cid * _HALF + p * r
            def _z1(k, _):
                pltpu.sync_copy(zrow, slab.at[pl.ds(sid * zsh + k, 1)])
                return 0
            lax.fori_loop(0, zsh, _z1, 0)
            plsc.subcore_barrier()

            for kk in range(fe // 16):
                z16 = jnp.zeros((16,), jnp.int32)
                d16 = jnp.full((16,), r, jnp.int32)
                fsb[pl.ds(kk * 16, 16)] = z16
                fdb[pl.ds(kk * 16, 16)] = d16
                fdd[pl.ds(kk * 16, 16)] = d16
            pltpu.async_copy(table.at[fsb], stgb, sem_g)
            pltpu.async_copy(stga, slab.at[fdd], sem_sa, add=True)

            pltpu.async_copy(srcv.at[pl.ds(ebase, _ECH)], src_c0, sem_c)
            pltpu.async_copy(dstv.at[pl.ds(ebase, _ECH)], dst_c0, sem_c)

            def scan_chunk(jc, carry, src_c, dst_c, src_n, dst_n):
                pltpu.make_async_copy(srcv.at[pl.ds(ebase, _ECH)],
                                      src_c, sem_c).wait()
                pltpu.make_async_copy(dstv.at[pl.ds(ebase, _ECH)],
                                      dst_c, sem_c).wait()
                nxt = jnp.minimum(ebase + (jc + 1) * _ECH, _E - _ECH)
                pltpu.async_copy(srcv.at[pl.ds(nxt, _ECH)], src_n, sem_c)
                pltpu.async_copy(dstv.at[pl.ds(nxt, _ECH)], dst_n, sem_c)

                def vec_body(jv, carry):
                    nf, nfire = carry
                    s16 = src_c[pl.ds(jv * 16, 16)]
                    d16 = dst_c[pl.ds(jv * 16, 16)]
                    dloc = d16 - wbase
                    m = dloc.astype(jnp.uint32) < jnp.uint32(r)
                    csum = jnp.cumsum(jnp.where(m, 1, 0))
                    pos = jnp.where(m, nf + csum - 1, trash)
                    plsc.store_scatter(psrc, [pos], s16)
                    plsc.store_scatter(pdst, [pos], dloc)
                    nf2 = nf + jnp.max(csum)

                    def do_fire(c):
                        v, nfr = c
                        fire_parity(nfr)
                        psrc[pl.ds(0, 16)] = psrc[pl.ds(fe, 16)]
                        pdst[pl.ds(0, 16)] = pdst[pl.ds(fe, 16)]
                        return v - fe, nfr + 1

                    return lax.cond(nf2 >= fe, do_fire, lambda c: c,
                                    (nf2, nfire))

                return lax.fori_loop(0, _ECH // 16, vec_body, carry)

            def chunk_body(jc, carry):
                return lax.cond(
                    jc % 2 == 0,
                    lambda c: scan_chunk(jc, c, src_c0, dst_c0,
                                         src_c1, dst_c1),
                    lambda c: scan_chunk(jc, c, src_c1, dst_c1,
                                         src_c0, dst_c0),
                    carry)

            nf, nfire = lax.fori_loop(0, _EPT // _ECH, chunk_body, (0, 0))
            pltpu.make_async_copy(srcv.at[pl.ds(ebase, _ECH)],
                                  src_c0, sem_c).wait()
            pltpu.make_async_copy(dstv.at[pl.ds(ebase, _ECH)],
                                  dst_c0, sem_c).wait()

            for kk in range(fe // 16):
                psrc[pl.ds(nf + kk * 16, 16)] = jnp.zeros((16,), jnp.int32)
                pdst[pl.ds(nf + kk * 16, 16)] = jnp.full((16,), r, jnp.int32)
            fire_parity(nfire)
            lax.cond(
                nfire % 2 == 0,
                lambda: (pltpu.make_async_copy(table.at[fsa], stga,
                                               sem_g).wait(),
                         pltpu.async_copy(stga, slab.at[fda], sem_sa,
                                          add=True))[0],
                lambda: (pltpu.make_async_copy(table.at[fsb], stgb,
                                               sem_g).wait(),
                         pltpu.async_copy(stgb, slab.at[fdb], sem_sb,
                                          add=True))[0],
            )
            pltpu.make_async_copy(stga, slab.at[fda], sem_sa).wait()
            pltpu.make_async_copy(stgb, slab.at[fdb], sem_sb).wait()
            plsc.subcore_barrier()

            pltpu.sync_copy(slab.at[pl.ds(sid * wsh, wsh)],
                            out.at[pl.ds(wbase + sid * wsh, wsh)])
            plsc.subcore_barrier()
            return 0

        lax.fori_loop(0, npass, pass_body, 0)

    return segsum


_segsum_l1 = _make_segsum(cm=2)
_segsum_l2 = _make_segsum(cm=8)

_ROWS_BLK = 400
_GRID = _N // _ROWS_BLK


def _elu(z):
    return jnp.where(z > 0, z, jnp.exp(jnp.minimum(z, 0.0)) - 1.0)


def _bdot(a, b):
    return jnp.dot(a.astype(jnp.bfloat16), b.astype(jnp.bfloat16),
                   preferred_element_type=jnp.float32)


def _tc1_body(s_ref, x_ref, wl_ref, b_ref, wr_ref, h_ref):
    s = s_ref[...]
    rcp = 1.0 / jnp.maximum(s[:, 128:129], 1.0)
    mean = s[:, :128] * rcp
    z = _bdot(mean, wl_ref[...]) + b_ref[...] + _bdot(x_ref[...], wr_ref[...])
    h_ref[...] = _elu(z)


def _tc1(sums1, x, W1l, b1, W1r):
    return pl.pallas_call(
        _tc1_body,
        grid=(_GRID,),
        in_specs=[
            pl.BlockSpec((_ROWS_BLK, 256), lambda i: (i, 0)),
            pl.BlockSpec((_ROWS_BLK, _D), lambda i: (i, 0)),
            pl.BlockSpec((_D, _H), lambda i: (0, 0)),
            pl.BlockSpec((1, _H), lambda i: (0, 0)),
            pl.BlockSpec((_D, _H), lambda i: (0, 0)),
        ],
        out_specs=pl.BlockSpec((_ROWS_BLK, _H), lambda i: (i, 0)),
        out_shape=jax.ShapeDtypeStruct((_N, _H), jnp.float32),
    )(sums1, x, W1l, b1, W1r)


def _tc2_body(s2_ref, s1_ref, h_ref, wl_ref, b_ref, wr_ref, wc_ref, bc_ref,
              o_ref):
    rcp = 1.0 / jnp.maximum(s1_ref[:, 128:129], 1.0)
    mean = s2_ref[...] * rcp
    z = _bdot(mean, wl_ref[...]) + b_ref[...] + _bdot(h_ref[...], wr_ref[...])
    h2 = _elu(z)
    o_ref[...] = _bdot(h2, wc_ref[...]) + bc_ref[...]


def _tc2(sums2, sums1, h, W2l, b2, W2r, Wcp, bcp):
    return pl.pallas_call(
        _tc2_body,
        grid=(_GRID,),
        in_specs=[
            pl.BlockSpec((_ROWS_BLK, _H), lambda i: (i, 0)),
            pl.BlockSpec((_ROWS_BLK, 256), lambda i: (i, 0)),
            pl.BlockSpec((_ROWS_BLK, _H), lambda i: (i, 0)),
            pl.BlockSpec((_H, _H), lambda i: (0, 0)),
            pl.BlockSpec((1, _H), lambda i: (0, 0)),
            pl.BlockSpec((_H, _H), lambda i: (0, 0)),
            pl.BlockSpec((_H, 256), lambda i: (0, 0)),
            pl.BlockSpec((1, 256), lambda i: (0, 0)),
        ],
        out_specs=pl.BlockSpec((_ROWS_BLK, 256), lambda i: (i, 0)),
        out_shape=jax.ShapeDtypeStruct((_N, 256), jnp.float32),
    )(sums2, sums1, h, W2l, b2, W2r, Wcp, bcp)


def kernel(x, edge_index, W1l, b1, W1r, W2l, b2, W2r, Wc, bc):
    src = edge_index[0].astype(jnp.int32)
    dst = edge_index[1].astype(jnp.int32)

    x_aug = jnp.concatenate(
        [x, jnp.ones((_N, 1), jnp.float32), jnp.zeros((_N, 127), jnp.float32)],
        axis=1)

    sums1 = _segsum_l1(x_aug.reshape(_N, 2, 128), src, dst)
    sums1 = sums1.reshape(_NP, 256)[:_N]
    h = _tc1(sums1, x, W1l, b1.reshape(1, _H), W1r)
    sums2 = _segsum_l2(h.reshape(_N, 8, 128), src, dst)
    sums2 = sums2.reshape(_NP, _H)[:_N]
    Wcp = jnp.pad(Wc, ((0, 0), (0, 256 - _C)))
    bcp = jnp.pad(bc, (0, 256 - _C)).reshape(1, 256)
    out = _tc2(sums2, sums1, h, W2l, b2.reshape(1, _H), W2r, Wcp, bcp)
    return out[:, :_C]
